# Initial kernel scaffold; baseline (speedup 1.0000x reference)
import jax
import jax.numpy as jnp
from jax.experimental import pallas as pl


def kernel(x, y, muK, cK):
    raise NotImplementedError("write your pallas kernel here")


# speedup vs baseline: 2.2352x; 2.2352x over previous
import jax
import jax.numpy as jnp
from jax import lax
from jax.experimental import pallas as pl
from jax.experimental.pallas import tpu as pltpu
from jax.experimental.pallas import tpu_sc as plsc

B, D, C = 16384, 512, 100000
L = 16
NW = 32
CPW = C // NW
NCHUNK = B // L
DCH = D // L


def _sc_update(x_hbm, y_hbm, mu_hbm, ck_hbm, mu_out, ck_out,
               y_v, sel_v, lbl_s, idx_s, r_s, ckn_s, ck_g,
               mu_rows, x_rows, sem):
    wid = lax.axis_index("s") * 2 + lax.axis_index("c")
    lo = wid * CPW
    hi = lo + CPW

    pltpu.sync_copy(y_hbm, y_v)

    lanes = lax.iota(jnp.int32, L)

    def sel_step(c, cnt):
        yv = y_v[pl.ds(c * L, L)]
        m = ((yv >= lo) & (yv < hi)).astype(jnp.int32)
        pos = jnp.where(m > 0, cnt + jnp.cumsum(m) - 1, B + L)
        plsc.store_scatter(sel_v, [pos], lanes + c * L)
        return cnt + jnp.sum(m)

    cnt = lax.fori_loop(0, NCHUNK, sel_step, jnp.int32(0))

    last = jnp.maximum(cnt - 1, 0)
    pad = plsc.load_gather(sel_v, [jnp.full((L,), 0, jnp.int32) + last])
    sel_v[pl.ds(cnt, L)] = pad
    ngroups = (cnt + (L - 1)) >> 4

    def group_step(g, carry):
        idx = sel_v[pl.ds(g * L, L)]
        lbl = plsc.load_gather(y_v, [idx])
        idx_s[...] = idx
        lbl_s[...] = lbl
        idx_eff = idx
        for s in range(1, L):
            perm = jnp.minimum(lanes + s, L - 1)
            rl = plsc.load_gather(lbl_s, [perm])
            ri = plsc.load_gather(idx_s, [perm])
            take = (rl == lbl) & (lanes < (L - s))
            idx_eff = jnp.where(take, ri, idx_eff)

        g1 = pltpu.async_copy(mu_hbm.at[lbl], mu_rows, sem)
        g2 = pltpu.async_copy(x_hbm.at[idx_eff], x_rows, sem)
        g3 = pltpu.async_copy(ck_hbm.at[lbl], ck_g, sem)
        g1.wait()
        g2.wait()
        g3.wait()

        ck1 = ck_g[...] + 1.0
        r_s[...] = 1.0 / ck1
        ckn_s[...] = ck1

        def row_step(j, c2):
            rj = plsc.load_gather(r_s, [jnp.full((L,), 0, jnp.int32) + j])
            for cpos in range(DCH):
                mu = mu_rows[j, pl.ds(cpos * L, L)]
                xx = x_rows[j, pl.ds(cpos * L, L)]
                mu_rows[j, pl.ds(cpos * L, L)] = mu + (xx - mu) * rj
            return c2

        lax.fori_loop(0, L, row_step, jnp.int32(0))

        s1 = pltpu.async_copy(mu_rows, mu_out.at[lbl], sem)
        s2 = pltpu.async_copy(ckn_s, ck_out.at[lbl], sem)
        s1.wait()
        s2.wait()
        return carry

    lax.fori_loop(0, ngroups, group_step, jnp.int32(0))


def kernel(x, y, muK, cK):
    mu_out = jax.new_ref(muK)
    ck_out = jax.new_ref(cK)
    mesh = plsc.VectorSubcoreMesh(core_axis_name="c", subcore_axis_name="s",
                                  num_cores=2, num_subcores=16)
    pl.kernel(
        _sc_update,
        out_type=(),
        mesh=mesh,
        compiler_params=pltpu.CompilerParams(needs_layout_passes=False),
        scratch_types=[
            pltpu.VMEM((B,), jnp.int32),
            pltpu.VMEM((B + 2 * L,), jnp.int32),
            pltpu.VMEM((L,), jnp.int32),
            pltpu.VMEM((L,), jnp.int32),
            pltpu.VMEM((L,), jnp.float32),
            pltpu.VMEM((L,), jnp.float32),
            pltpu.VMEM((L,), jnp.float32),
            pltpu.VMEM((L, D), jnp.float32),
            pltpu.VMEM((L, D), jnp.float32),
            pltpu.SemaphoreType.DMA,
        ],
    )(x, y, muK, cK, mu_out, ck_out)
    return mu_out[...], ck_out[...]


# speedup vs baseline: 2.5567x; 1.1438x over previous
import jax
import jax.numpy as jnp
from jax import lax
from jax.experimental import pallas as pl
from jax.experimental.pallas import tpu as pltpu
from jax.experimental.pallas import tpu_sc as plsc

B, D, C = 16384, 512, 100000
L = 16
NW = 32
CPW = C // NW
NCHUNK = B // L
DCH = D // L
NBUF = 3


def _sc_update(x_hbm, y_hbm, mu_hbm, ck_hbm, mu_out, ck_out,
               y_v, sel_v, lbl_s, idx_s, r_s,
               lblv0, lblv1, lblv2, mur0, mur1, mur2, xr0, xr1, xr2,
               ckg0, ckg1, ckg2, ckn0, ckn1, ckn2,
               gsem0, gsem1, gsem2, ssem0, ssem1, ssem2):
    buf = (
        (lblv0, mur0, xr0, ckg0, ckn0, gsem0, ssem0),
        (lblv1, mur1, xr1, ckg1, ckn1, gsem1, ssem1),
        (lblv2, mur2, xr2, ckg2, ckn2, gsem2, ssem2),
    )
    wid = lax.axis_index("s") * 2 + lax.axis_index("c")
    lo = wid * CPW
    hi = lo + CPW

    pltpu.sync_copy(y_hbm, y_v)

    lanes = lax.iota(jnp.int32, L)

    def sel_step(c, cnt):
        yv = y_v[pl.ds(c * L, L)]
        m = ((yv >= lo) & (yv < hi)).astype(jnp.int32)
        pos = jnp.where(m > 0, cnt + jnp.cumsum(m) - 1, B + L)
        plsc.store_scatter(sel_v, [pos], lanes + c * L)
        return cnt + jnp.sum(m)

    cnt = lax.fori_loop(0, NCHUNK, sel_step, jnp.int32(0))

    last = jnp.maximum(cnt - 1, 0)
    pad = plsc.load_gather(sel_v, [jnp.full((L,), 0, jnp.int32) + last])
    sel_v[pl.ds(cnt, L)] = pad
    ngroups = (cnt + (L - 1)) >> 4

    def issue_gathers(g, k):
        lblv, mur, xr, ckg, _, gsem, _ = buf[k]
        idx = sel_v[pl.ds(g * L, L)]
        lbl = plsc.load_gather(y_v, [idx])
        idx_s[...] = idx
        lbl_s[...] = lbl
        idx_eff = idx
        for s in range(1, L):
            perm = jnp.minimum(lanes + s, L - 1)
            rl = plsc.load_gather(lbl_s, [perm])
            ri = plsc.load_gather(idx_s, [perm])
            take = (rl == lbl) & (lanes < (L - s))
            idx_eff = jnp.where(take, ri, idx_eff)
        lblv[...] = lbl
        pltpu.async_copy(mu_hbm.at[lbl], mur, gsem)
        pltpu.async_copy(x_hbm.at[idx_eff], xr, gsem)
        pltpu.async_copy(ck_hbm.at[lbl], ckg, gsem)

    def process(g, k):
        lblv, mur, xr, ckg, ckn, gsem, _ = buf[k]
        nk = (k + 1) % NBUF
        pk = (k + 2) % NBUF
        lbl = lblv[...]
        pltpu.make_async_copy(mu_hbm.at[lbl], mur, gsem).wait()
        pltpu.make_async_copy(x_hbm.at[lbl], xr, gsem).wait()
        pltpu.make_async_copy(ck_hbm.at[lbl], ckg, gsem).wait()

        @pl.when(g + 1 < ngroups)
        def _():
            issue_gathers(g + 1, nk)

        ck1 = ckg[...] + 1.0
        r_s[...] = 1.0 / ck1
        ckn[...] = ck1

        def row_step(j, c2):
            rj = plsc.load_gather(r_s, [jnp.full((L,), 0, jnp.int32) + j])
            for cpos in range(DCH):
                mu = mur[j, pl.ds(cpos * L, L)]
                xx = xr[j, pl.ds(cpos * L, L)]
                mur[j, pl.ds(cpos * L, L)] = mu + (xx - mu) * rj
            return c2

        lax.fori_loop(0, L, row_step, jnp.int32(0))

        @pl.when(g > 0)
        def _():
            plbl, pmur, _, _, pckn, _, pssem = buf[pk]
            lp = plbl[...]
            pltpu.make_async_copy(pmur, mu_out.at[lp], pssem).wait()
            pltpu.make_async_copy(pckn, ck_out.at[lp], pssem).wait()

        ssem = buf[k][6]
        pltpu.async_copy(mur, mu_out.at[lbl], ssem)
        pltpu.async_copy(ckn, ck_out.at[lbl], ssem)

    @pl.when(ngroups > 0)
    def _():
        issue_gathers(0, 0)

    def tri_step(p, carry):
        for k in range(NBUF):
            g = p * NBUF + k

            @pl.when(g < ngroups)
            def _(g=g, k=k):
                process(g, k)

        return carry

    lax.fori_loop(0, (ngroups + (NBUF - 1)) // NBUF, tri_step, jnp.int32(0))

    for k in range(NBUF):
        @pl.when((ngroups > 0) & ((ngroups - 1) % NBUF == k))
        def _(k=k):
            lblv, mur, _, _, ckn, _, ssem = buf[k]
            lp = lblv[...]
            pltpu.make_async_copy(mur, mu_out.at[lp], ssem).wait()
            pltpu.make_async_copy(ckn, ck_out.at[lp], ssem).wait()


def kernel(x, y, muK, cK):
    mu_out = jax.new_ref(muK)
    ck_out = jax.new_ref(cK)
    mesh = plsc.VectorSubcoreMesh(core_axis_name="c", subcore_axis_name="s",
                                  num_cores=2, num_subcores=16)
    vec16i = pltpu.VMEM((L,), jnp.int32)
    vec16f = pltpu.VMEM((L,), jnp.float32)
    rows = pltpu.VMEM((L, D), jnp.float32)
    pl.kernel(
        _sc_update,
        out_type=(),
        mesh=mesh,
        compiler_params=pltpu.CompilerParams(needs_layout_passes=False),
        scratch_types=[
            pltpu.VMEM((B,), jnp.int32),
            pltpu.VMEM((B + 2 * L,), jnp.int32),
            vec16i, vec16i, vec16f,
            vec16i, vec16i, vec16i,
            rows, rows, rows,
            rows, rows, rows,
            vec16f, vec16f, vec16f,
            vec16f, vec16f, vec16f,
            pltpu.SemaphoreType.DMA, pltpu.SemaphoreType.DMA,
            pltpu.SemaphoreType.DMA, pltpu.SemaphoreType.DMA,
            pltpu.SemaphoreType.DMA, pltpu.SemaphoreType.DMA,
        ],
    )(x, y, muK, cK, mu_out, ck_out)
    return mu_out[...], ck_out[...]
